# Initial kernel scaffold; baseline (speedup 1.0000x reference)
import jax
import jax.numpy as jnp
from jax.experimental import pallas as pl


def kernel(embed, onehot_label):
    raise NotImplementedError("write your pallas kernel here")


# speedup vs baseline: 1.0880x; 1.0880x over previous
import contextlib
import functools

import jax
import jax.numpy as jnp
import numpy as np
from jax import lax
from jax.experimental import pallas as pl
from jax.experimental.pallas import tpu as pltpu
from jax.experimental.pallas import tpu_sc as plsc

_B = 16384
_D = 512
_SUB = 128
_N = _D // _SUB
_L = 1000
_NC, _NS = 2, 16
_NW = _NC * _NS
_CBE = 128
_CBL = 64


def _pack(a, cb):
    k = max(1, -(-len(a) // (_NW * cb)))
    pad = _NW * cb * k - len(a)
    a2 = np.concatenate([a, np.full(pad, a[0], a.dtype)])
    return np.ascontiguousarray(a2.reshape(_NW, k, cb).astype(np.int32)), k


def _tf2x32(k1, k2, x1, x2):
    def rotl(x, d):
        return (x << np.uint32(d)) | (x >> np.uint32(32 - d))

    def rnds(x0, x1v, rots):
        for r in rots:
            x0 = x0 + x1v
            x1v = rotl(x1v, r)
            x1v = x0 ^ x1v
        return x0, x1v

    r0, r1 = (13, 15, 26, 6), (17, 29, 16, 24)
    ks2 = k1 ^ k2 ^ np.uint32(0x1BD11BDA)
    x0, x1v = x1 + k1, x2 + k2
    x0, x1v = rnds(x0, x1v, r0)
    x0, x1v = x0 + k2, x1v + ks2 + np.uint32(1)
    x0, x1v = rnds(x0, x1v, r1)
    x0, x1v = x0 + ks2, x1v + k1 + np.uint32(2)
    x0, x1v = rnds(x0, x1v, r0)
    x0, x1v = x0 + k1, x1v + k2 + np.uint32(3)
    x0, x1v = rnds(x0, x1v, r1)
    x0, x1v = x0 + k2, x1v + ks2 + np.uint32(4)
    x0, x1v = rnds(x0, x1v, r0)
    return x0 + ks2, x1v + k1 + np.uint32(5)


def _np_fold_in(key, data):
    a, b = _tf2x32(key[0], key[1], np.uint32([0]), np.uint32([data]))
    return np.array([a[0], b[0]], np.uint32)


def _np_random_bits(key, n):
    b1, b2 = _tf2x32(key[0], key[1], np.zeros(n, np.uint32), np.arange(n, dtype=np.uint32))
    return b1 ^ b2


def _np_split(key):
    b1, b2 = _tf2x32(key[0], key[1], np.uint32([0, 0]), np.uint32([0, 1]))
    return (np.array([b1[0], b2[0]], np.uint32), np.array([b1[1], b2[1]], np.uint32))


def _np_uniform01(key, n):
    bits = _np_random_bits(key, n)
    fb = (bits >> np.uint32(9)) | np.uint32(0x3F800000)
    return fb.view(np.float32) - np.float32(1.0)


def _np_permutation(key, n):
    x = np.arange(n, dtype=np.int64)
    exponent = 3
    num_rounds = int(np.ceil(exponent * np.log(max(1, n)) / np.log(np.iinfo(np.uint32).max)))
    for _ in range(num_rounds):
        key, subkey = _np_split(key)
        sort_keys = _np_random_bits(subkey, n)
        x = x[np.argsort(sort_keys, kind="stable")]
    return x


def _draw_dec_perms():
    key = np.array([0, 42], np.uint32)
    dec = _np_uniform01(_np_fold_in(key, 0), _B) < 0.5
    perms = [np.arange(_B, dtype=np.int64)]
    for i in range(1, _N):
        perms.append(_np_permutation(_np_fold_in(key, i), _B))
    return dec, perms


def _build_consts():
    dec, perms = _draw_dec_perms()
    ar = np.arange(_B, dtype=np.int64)

    srcs = np.stack([np.where(dec if i else np.zeros(_B, bool), perms[i], ar) for i in range(_N)], axis=1)
    eidx = (4 * srcs + np.arange(_N)[None, :]).reshape(-1)
    eidx_p, ke = _pack(eidx, _CBE)

    rows = ar
    ndec = ~dec
    c_src = rows[ndec]
    c_dst = 5 * rows[ndec]
    q_src = np.concatenate([perms[j - 1][dec] for j in range(1, 5)])
    q_dst = np.concatenate([5 * rows[dec] + j for j in range(1, 5)])
    z_dst = np.concatenate([5 * rows[dec]] + [5 * rows[ndec] + j for j in range(1, 5)])

    csrc_p, kc = _pack(c_src, _CBL)
    cdst_p, _ = _pack(c_dst, _CBL)
    qsrc_p, kq = _pack(q_src, _CBL)
    qdst_p, _ = _pack(q_dst, _CBL)
    zdst_p, kz = _pack(z_dst, _CBL)
    return (eidx_p, csrc_p, cdst_p, qsrc_p, qdst_p, zdst_p), (ke, kc, kq, kz)


_CONSTS, (_KE, _KC, _KQ, _KZ) = _build_consts()


def _body(embed_h, label_h, eidx_h, csrc_h, cdst_h, qsrc_h, qdst_h, zdst_h,
          out_e, out_l, eiv, ebuf, siv, div, lbuf, sem):
    w = lax.axis_index("c") * _NS + lax.axis_index("s")

    def ebody(k, _):
        pltpu.sync_copy(eidx_h.at[w, k], eiv)
        pltpu.async_copy(embed_h.at[eiv], ebuf, sem).wait()
        pltpu.sync_copy(ebuf, out_e.at[pl.ds((w * _KE + k) * _CBE, _CBE)])
        return _

    lax.fori_loop(0, _KE, ebody, None)

    zv = jnp.zeros((16,), jnp.float32)

    def zrow(r, _):
        for t in range(62):
            lbuf[r, pl.ds(t * 16, 16)] = zv
        lbuf[r, pl.ds(984, 16)] = zv
        return _

    lax.fori_loop(0, _CBL, zrow, None)

    def zbody(k, _):
        pltpu.sync_copy(zdst_h.at[w, k], div)
        pltpu.async_copy(lbuf, out_l.at[div], sem).wait()
        return _

    lax.fori_loop(0, _KZ, zbody, None)

    def cbody(k, _):
        pltpu.sync_copy(csrc_h.at[w, k], siv)
        pltpu.sync_copy(cdst_h.at[w, k], div)
        pltpu.async_copy(label_h.at[siv], lbuf, sem).wait()
        pltpu.async_copy(lbuf, out_l.at[div], sem).wait()
        return _

    lax.fori_loop(0, _KC, cbody, None)

    qs = jnp.full((16,), 0.25, jnp.float32)
    qt = jnp.where(lax.iota(jnp.int32, 16) < 8, 1.0, 0.25).astype(jnp.float32)

    def qbody(k, _):
        pltpu.sync_copy(qsrc_h.at[w, k], siv)
        pltpu.sync_copy(qdst_h.at[w, k], div)
        pltpu.async_copy(label_h.at[siv], lbuf, sem).wait()

        def srow(r, _2):
            for t in range(62):
                lbuf[r, pl.ds(t * 16, 16)] = lbuf[r, pl.ds(t * 16, 16)] * qs
            lbuf[r, pl.ds(984, 16)] = lbuf[r, pl.ds(984, 16)] * qt
            return _2

        lax.fori_loop(0, _CBL, srow, None)
        pltpu.async_copy(lbuf, out_l.at[div], sem).wait()
        return _

    lax.fori_loop(0, _KQ, qbody, None)


@functools.cache
def _sc_call():
    return pl.kernel(
        _body,
        out_type=(
            jax.ShapeDtypeStruct((_B * _N, _SUB), jnp.float32),
            jax.ShapeDtypeStruct((_B * 5, _L), jnp.float32),
        ),
        mesh=plsc.VectorSubcoreMesh(
            core_axis_name="c", subcore_axis_name="s", num_cores=_NC, num_subcores=_NS
        ),
        compiler_params=pltpu.CompilerParams(use_tc_tiling_on_sc=False),
        scratch_types=[
            pltpu.VMEM((_CBE,), jnp.int32),
            pltpu.VMEM((_CBE, _SUB), jnp.float32),
            pltpu.VMEM((_CBL,), jnp.int32),
            pltpu.VMEM((_CBL,), jnp.int32),
            pltpu.VMEM((_CBL, _L), jnp.float32),
            pltpu.SemaphoreType.DMA,
        ],
    )


def kernel(embed, onehot_label):
    embed_flat = embed.reshape(_B * _N, _SUB)
    consts = [jnp.asarray(c) for c in _CONSTS]
    out_e, out_l = _sc_call()(embed_flat, onehot_label, *consts)
    return out_e.reshape(_B, _D), out_l.reshape(_B, 5 * _L)


# speedup vs baseline: 1.1364x; 1.0445x over previous
import functools

import jax
import jax.numpy as jnp
import numpy as np
from jax import lax
from jax.experimental import pallas as pl
from jax.experimental.pallas import tpu as pltpu
from jax.experimental.pallas import tpu_sc as plsc

_B = 16384
_D = 512
_SUB = 128
_N = _D // _SUB
_L = 1000
_NC, _NS = 2, 16
_NW = _NC * _NS
_CBE = 128
_KE = _B * _N // (_NW * _CBE)
_CBL = 16
_KC = 16
_KQ = 68
_KZ = 80


def _tf2x32(k1, k2, x1, x2):
    def rotl(x, d):
        return (x << np.uint32(d)) | (x >> np.uint32(32 - d))

    def rnds(x0, x1v, rots):
        for r in rots:
            x0 = x0 + x1v
            x1v = rotl(x1v, r)
            x1v = x0 ^ x1v
        return x0, x1v

    r0, r1 = (13, 15, 26, 6), (17, 29, 16, 24)
    ks2 = k1 ^ k2 ^ np.uint32(0x1BD11BDA)
    x0, x1v = x1 + k1, x2 + k2
    x0, x1v = rnds(x0, x1v, r0)
    x0, x1v = x0 + k2, x1v + ks2 + np.uint32(1)
    x0, x1v = rnds(x0, x1v, r1)
    x0, x1v = x0 + ks2, x1v + k1 + np.uint32(2)
    x0, x1v = rnds(x0, x1v, r0)
    x0, x1v = x0 + k1, x1v + k2 + np.uint32(3)
    x0, x1v = rnds(x0, x1v, r1)
    x0, x1v = x0 + k2, x1v + ks2 + np.uint32(4)
    x0, x1v = rnds(x0, x1v, r0)
    return x0 + ks2, x1v + k1 + np.uint32(5)


def _np_fold_in(key, data):
    a, b = _tf2x32(key[0], key[1], np.uint32([0]), np.uint32([data]))
    return np.array([a[0], b[0]], np.uint32)


def _np_random_bits(key, n):
    b1, b2 = _tf2x32(key[0], key[1], np.zeros(n, np.uint32), np.arange(n, dtype=np.uint32))
    return b1 ^ b2


def _np_split(key):
    b1, b2 = _tf2x32(key[0], key[1], np.uint32([0, 0]), np.uint32([0, 1]))
    return (np.array([b1[0], b2[0]], np.uint32), np.array([b1[1], b2[1]], np.uint32))


def _np_uniform01(key, n):
    bits = _np_random_bits(key, n)
    fb = (bits >> np.uint32(9)) | np.uint32(0x3F800000)
    return fb.view(np.float32) - np.float32(1.0)


def _np_permutation(key, n):
    x = np.arange(n, dtype=np.int64)
    exponent = 3
    num_rounds = int(np.ceil(exponent * np.log(max(1, n)) / np.log(np.iinfo(np.uint32).max)))
    for _ in range(num_rounds):
        key, subkey = _np_split(key)
        sort_keys = _np_random_bits(subkey, n)
        x = x[np.argsort(sort_keys, kind="stable")]
    return x


def _pack(a, k):
    n = _NW * k * _CBL
    assert len(a) <= n, (len(a), n)
    a2 = np.concatenate([a, np.full(n - len(a), a[0], a.dtype)])
    return np.ascontiguousarray(a2.reshape(_NW, k, _CBL).astype(np.int32))


def _build_consts():
    key = np.array([0, 42], np.uint32)
    dec = _np_uniform01(_np_fold_in(key, 0), _B) < 0.5
    perms = [np.arange(_B, dtype=np.int64)]
    for i in range(1, _N):
        perms.append(_np_permutation(_np_fold_in(key, i), _B))
    ar = np.arange(_B, dtype=np.int64)
    ndec = ~dec

    srcs = np.stack([np.where(dec if i else np.zeros(_B, bool), perms[i], ar)
                     for i in range(_N)], axis=1)
    eidx = (4 * srcs + np.arange(_N)[None, :]).reshape(-1)
    eidx_p = np.ascontiguousarray(eidx.reshape(_NW, _KE, _CBE).astype(np.int32))

    c_src = ar[ndec]
    c_dst = 5 * ar[ndec]
    q_src = np.concatenate([perms[j - 1][dec] for j in range(1, 5)])
    q_dst = np.concatenate([5 * ar[dec] + j for j in range(1, 5)])
    z_dst = np.concatenate([5 * ar[dec]] + [5 * ar[ndec] + j for j in range(1, 5)])

    return (eidx_p, _pack(c_src, _KC), _pack(c_dst, _KC),
            _pack(q_src, _KQ), _pack(q_dst, _KQ), _pack(z_dst, _KZ))


_CONSTS = _build_consts()


def _body(embed_h, label_h, eidx_h, csrc_h, cdst_h, qsrc_h, qdst_h, zdst_h,
          out_e, out_l, eiv, csv, cdv, qsv, qdv, zdv, eb, lb, zb,
          g0, g1, g2, g3, s0, s1, s2, s3, zsem):
    w = lax.axis_index("c") * _NS + lax.axis_index("s")
    gsems = (g0, g1, g2, g3)
    ssems = (s0, s1, s2, s3)

    pltpu.sync_copy(eidx_h.at[w], eiv)
    pltpu.sync_copy(csrc_h.at[w], csv)
    pltpu.sync_copy(cdst_h.at[w], cdv)
    pltpu.sync_copy(qsrc_h.at[w], qsv)
    pltpu.sync_copy(qdst_h.at[w], qdv)
    pltpu.sync_copy(zdst_h.at[w], zdv)

    zv = jnp.zeros((16,), jnp.float32)

    def zrow(r, _):
        for t in range(62):
            zb[r, pl.ds(t * 16, 16)] = zv
        zb[r, pl.ds(984, 16)] = zv
        return _

    lax.fori_loop(0, _CBL, zrow, None)

    def zgroup(g, _):
        hs = [pltpu.async_copy(zb, out_l.at[zdv.at[g * 8 + t]], zsem)
              for t in range(8)]
        for h in hs:
            h.wait()
        return _

    lax.fori_loop(0, _KZ // 8, zgroup, None)

    def ebody(i, _):
        hs, ss = [], []
        for t in range(2):
            k = i * 2 + t
            hs.append(pltpu.async_copy(embed_h.at[eiv.at[k]], eb.at[t], gsems[t]))
        for t in range(2):
            k = i * 2 + t
            hs[t].wait()
            base = (w * _KE + k) * _CBE
            ss.append(pltpu.async_copy(eb.at[t], out_e.at[pl.ds(base, _CBE)], ssems[t]))
        for s in ss:
            s.wait()
        return _

    lax.fori_loop(0, _KE // 2, ebody, None)

    def cbody(i, _):
        hs, ss = [], []
        for t in range(4):
            k = i * 4 + t
            hs.append(pltpu.async_copy(label_h.at[csv.at[k]], lb.at[t], gsems[t]))
        for t in range(4):
            k = i * 4 + t
            hs[t].wait()
            ss.append(pltpu.async_copy(lb.at[t], out_l.at[cdv.at[k]], ssems[t]))
        for s in ss:
            s.wait()
        return _

    lax.fori_loop(0, _KC // 4, cbody, None)

    qs = jnp.full((16,), 0.25, jnp.float32)
    qt = jnp.where(lax.iota(jnp.int32, 16) < 8, 1.0, 0.25).astype(jnp.float32)

    def qbody(i, _):
        hs, ss = [], []
        for t in range(4):
            k = i * 4 + t
            hs.append(pltpu.async_copy(label_h.at[qsv.at[k]], lb.at[t], gsems[t]))
        for t in range(4):
            k = i * 4 + t
            hs[t].wait()

            def srow(r, _2, t=t):
                for c in range(62):
                    lb[t, r, pl.ds(c * 16, 16)] = lb[t, r, pl.ds(c * 16, 16)] * qs
                lb[t, r, pl.ds(984, 16)] = lb[t, r, pl.ds(984, 16)] * qt
                return _2

            lax.fori_loop(0, _CBL, srow, None)
            ss.append(pltpu.async_copy(lb.at[t], out_l.at[qdv.at[k]], ssems[t]))
        for s in ss:
            s.wait()
        return _

    lax.fori_loop(0, _KQ // 4, qbody, None)


@functools.cache
def _sc_call():
    return pl.kernel(
        _body,
        out_type=(
            jax.ShapeDtypeStruct((_B * _N, _SUB), jnp.float32),
            jax.ShapeDtypeStruct((_B * 5, _L), jnp.float32),
        ),
        mesh=plsc.VectorSubcoreMesh(
            core_axis_name="c", subcore_axis_name="s", num_cores=_NC, num_subcores=_NS
        ),
        compiler_params=pltpu.CompilerParams(use_tc_tiling_on_sc=False),
        scratch_types=[
            pltpu.VMEM((_KE, _CBE), jnp.int32),
            pltpu.VMEM((_KC, _CBL), jnp.int32),
            pltpu.VMEM((_KC, _CBL), jnp.int32),
            pltpu.VMEM((_KQ, _CBL), jnp.int32),
            pltpu.VMEM((_KQ, _CBL), jnp.int32),
            pltpu.VMEM((_KZ, _CBL), jnp.int32),
            pltpu.VMEM((2, _CBE, _SUB), jnp.float32),
            pltpu.VMEM((4, _CBL, _L), jnp.float32),
            pltpu.VMEM((_CBL, _L), jnp.float32),
        ] + [pltpu.SemaphoreType.DMA] * 9,
    )


def kernel(embed, onehot_label):
    embed_flat = embed.reshape(_B * _N, _SUB)
    consts = [jnp.asarray(c) for c in _CONSTS]
    out_e, out_l = _sc_call()(embed_flat, onehot_label, *consts)
    return out_e.reshape(_B, _D), out_l.reshape(_B, 5 * _L)
